# Initial kernel scaffold; baseline (speedup 1.0000x reference)
import functools

import jax
import jax.numpy as jnp
from jax import lax
from jax.experimental import pallas as pl
from jax.experimental.pallas import tpu as pltpu
from jax.experimental.pallas import tpu_sc as plsc

N_NODES = 10000
N_EDGES = 320000
D_EDGE = 16

NC = 2
NS = 16
NW = NC * NS

ROW = 125
N_ROWS = N_EDGES // ROW
ROWS_PER_TILE = N_ROWS // NW
K = 16
STEPS = ROWS_PER_TILE // K
NODES_PER_SUB = N_NODES // NS


def _sc_body(zeros_hbm, idx_hbm, attr_hbm, out_hbm, idx_v, attr_v, acc):
    cid = lax.axis_index("c")
    sid = lax.axis_index("s")
    wid = sid * NC + cid

    stripe = pl.ds(sid * NODES_PER_SUB, NODES_PER_SUB)
    pltpu.sync_copy(zeros_hbm.at[stripe], acc.at[stripe])
    plsc.subcore_barrier()

    row0 = wid * ROWS_PER_TILE

    def step(s, carry):
        base = row0 + s * K
        pltpu.sync_copy(idx_hbm.at[pl.ds(base, K)], idx_v)
        pltpu.sync_copy(attr_hbm.at[pl.ds(base, K)], attr_v)
        for j in range(K):
            pltpu.sync_copy(attr_v.at[j], acc.at[idx_v.at[j]], add=True)
        return carry

    lax.fori_loop(0, STEPS, step, 0)
    plsc.subcore_barrier()

    pltpu.sync_copy(acc.at[stripe], out_hbm.at[cid, stripe])


_sc_scatter = functools.partial(
    pl.kernel,
    mesh=plsc.VectorSubcoreMesh(core_axis_name="c", subcore_axis_name="s"),
    out_type=jax.ShapeDtypeStruct((NC, N_NODES, D_EDGE), jnp.float32),
    scratch_types=[
        pltpu.VMEM((K, ROW), jnp.int32),
        pltpu.VMEM((K, ROW, D_EDGE), jnp.float32),
        pltpu.VMEM_SHARED((N_NODES, D_EDGE), jnp.float32),
    ],
)(_sc_body)


def _combine_body(p_ref, o_ref):
    o_ref[...] = p_ref[0] + p_ref[1]


@jax.jit
def kernel(x, edge_index, edge_attr):
    del x
    dst = edge_index[1].astype(jnp.int32).reshape(N_ROWS, ROW)
    attr = edge_attr.reshape(N_ROWS, ROW, D_EDGE)
    zeros = jnp.zeros((N_NODES, D_EDGE), jnp.float32)
    partials = _sc_scatter(zeros, dst, attr)
    out = pl.pallas_call(
        _combine_body,
        out_shape=jax.ShapeDtypeStruct((N_NODES * D_EDGE // 128, 128), jnp.float32),
    )(partials.reshape(NC, N_NODES * D_EDGE // 128, 128))
    return out.reshape(N_NODES, D_EDGE)


# speedup vs baseline: 4.7522x; 4.7522x over previous
import functools

import jax
import jax.numpy as jnp
from jax import lax
from jax.experimental import pallas as pl
from jax.experimental.pallas import tpu as pltpu
from jax.experimental.pallas import tpu_sc as plsc

N_NODES = 10000
N_EDGES = 320000
D_EDGE = 16

NC = 2
NS = 16
NW = NC * NS

ROW = 125
N_ROWS = N_EDGES // ROW
ROWS_PER_TILE = N_ROWS // NW
K = 16
STEPS = ROWS_PER_TILE // K
STRIPE = 624
REM_START = STRIPE * NS
REM = N_NODES - REM_START


def _sc_body(zeros_hbm, idx_hbm, attr_hbm, out_hbm, idx_v, attr_v, acc):
    cid = lax.axis_index("c")
    sid = lax.axis_index("s")
    wid = sid * NC + cid

    stripe = pl.ds(sid * STRIPE, STRIPE)
    rem = pl.ds(REM_START, REM)
    pltpu.sync_copy(zeros_hbm.at[stripe], acc.at[stripe])

    @pl.when(sid == 0)
    def _zero_rem():
        pltpu.sync_copy(zeros_hbm.at[rem], acc.at[rem])

    plsc.subcore_barrier()

    row0 = wid * ROWS_PER_TILE

    def step(s, carry):
        base = row0 + s * K
        pltpu.sync_copy(idx_hbm.at[pl.ds(base, K)], idx_v)
        pltpu.sync_copy(attr_hbm.at[pl.ds(base, K)], attr_v)
        for j in range(K):
            pltpu.sync_copy(attr_v.at[j], acc.at[idx_v.at[j]], add=True)
        return carry

    lax.fori_loop(0, STEPS, step, 0)
    plsc.subcore_barrier()

    pltpu.sync_copy(acc.at[stripe], out_hbm.at[cid, stripe])

    @pl.when(sid == 0)
    def _out_rem():
        pltpu.sync_copy(acc.at[rem], out_hbm.at[cid, rem])


_sc_scatter = functools.partial(
    pl.kernel,
    mesh=plsc.VectorSubcoreMesh(core_axis_name="c", subcore_axis_name="s"),
    out_type=jax.ShapeDtypeStruct((NC, N_NODES, D_EDGE), jnp.float32),
    scratch_types=[
        pltpu.VMEM((K, ROW), jnp.int32),
        pltpu.VMEM((K, ROW, D_EDGE), jnp.float32),
        pltpu.VMEM_SHARED((N_NODES, D_EDGE), jnp.float32),
    ],
    compiler_params=pltpu.CompilerParams(use_tc_tiling_on_sc=False),
)(_sc_body)


def _combine_body(p_ref, o_ref):
    o_ref[...] = p_ref[0] + p_ref[1]


@jax.jit
def kernel(x, edge_index, edge_attr):
    del x
    dst = edge_index[1].astype(jnp.int32).reshape(N_ROWS, ROW)
    attr = edge_attr.reshape(N_ROWS, ROW, D_EDGE)
    zeros = jnp.zeros((N_NODES, D_EDGE), jnp.float32)
    partials = _sc_scatter(zeros, dst, attr)
    out = pl.pallas_call(
        _combine_body,
        out_shape=jax.ShapeDtypeStruct((N_NODES * D_EDGE // 128, 128), jnp.float32),
    )(partials.reshape(NC, N_NODES * D_EDGE // 128, 128))
    return out.reshape(N_NODES, D_EDGE)


# speedup vs baseline: 5.4515x; 1.1472x over previous
import functools

import jax
import jax.numpy as jnp
from jax import lax
from jax.experimental import pallas as pl
from jax.experimental.pallas import tpu as pltpu
from jax.experimental.pallas import tpu_sc as plsc

N_NODES = 10000
N_EDGES = 320000
D_EDGE = 16

NC = 2
NS = 16
NW = NC * NS

ROW = 128
N_CHUNKS = N_EDGES // ROW
CPT = N_CHUNKS // NW
REM_CHUNKS = N_CHUNKS - CPT * NW
K = 13
STEPS = CPT // K
CHUNK = K * ROW
N_PAD = 10240
STRIPE = N_PAD // NS


def _sc_body(zeros_hbm, idx_hbm, attr_hbm, out_hbm, idx_v, attr_v, acc):
    cid = lax.axis_index("c")
    sid = lax.axis_index("s")
    wid = sid * NC + cid

    stripe = pl.ds(sid * STRIPE, STRIPE)
    pltpu.sync_copy(zeros_hbm.at[stripe], acc.at[stripe])
    plsc.subcore_barrier()

    e0 = wid * CPT * ROW

    def step(s, carry):
        base = e0 + s * CHUNK
        pltpu.sync_copy(idx_hbm.at[1, pl.ds(base, CHUNK)], idx_v)
        pltpu.sync_copy(attr_hbm.at[pl.ds(base, CHUNK)], attr_v)
        for j in range(K):
            pltpu.sync_copy(
                attr_v.at[pl.ds(j * ROW, ROW)],
                acc.at[idx_v.at[pl.ds(j * ROW, ROW)]],
                add=True,
            )
        return carry

    lax.fori_loop(0, STEPS, step, 0)

    @pl.when(wid < REM_CHUNKS)
    def _rem_chunk():
        base = (CPT * NW + wid) * ROW
        pltpu.sync_copy(idx_hbm.at[1, pl.ds(base, ROW)], idx_v.at[pl.ds(0, ROW)])
        pltpu.sync_copy(attr_hbm.at[pl.ds(base, ROW)], attr_v.at[pl.ds(0, ROW)])
        pltpu.sync_copy(
            attr_v.at[pl.ds(0, ROW)],
            acc.at[idx_v.at[pl.ds(0, ROW)]],
            add=True,
        )

    plsc.subcore_barrier()

    pltpu.sync_copy(acc.at[stripe], out_hbm.at[cid, stripe])


_sc_scatter = functools.partial(
    pl.kernel,
    mesh=plsc.VectorSubcoreMesh(core_axis_name="c", subcore_axis_name="s"),
    out_type=jax.ShapeDtypeStruct((NC, N_PAD, D_EDGE), jnp.float32),
    scratch_types=[
        pltpu.VMEM((CHUNK,), jnp.int32),
        pltpu.VMEM((CHUNK, D_EDGE), jnp.float32),
        pltpu.VMEM_SHARED((N_PAD, D_EDGE), jnp.float32),
    ],
    compiler_params=pltpu.CompilerParams(use_tc_tiling_on_sc=False),
)(_sc_body)


def _combine_body(p_ref, o_ref):
    o_ref[...] = p_ref[0] + p_ref[1]


@jax.jit
def kernel(x, edge_index, edge_attr):
    del x
    idx = edge_index.astype(jnp.int32)
    zeros = jnp.zeros((N_PAD, D_EDGE), jnp.float32)
    partials = _sc_scatter(zeros, idx, edge_attr)
    combined = pl.pallas_call(
        _combine_body,
        out_shape=jax.ShapeDtypeStruct((N_PAD * D_EDGE // 128, 128), jnp.float32),
    )(partials.reshape(NC, N_PAD * D_EDGE // 128, 128))
    return combined.reshape(N_PAD, D_EDGE)[:N_NODES]
